# baseline (speedup 1.0000x reference)
import functools

import jax
import jax.numpy as jnp
from jax import lax
from jax.experimental import pallas as pl
from jax.experimental.pallas import tpu as pltpu
from jax.experimental.pallas import tpu_sc as plsc

N_NODES = 10000
D = 128
N_EDGES = 320000

NC = 2
NS = 16
NW = NC * NS
E_PER_TILE = N_EDGES // NW
EB = 125
NB = E_PER_TILE // EB
INNER = 5
OUTER = NB // INNER

N_PAD = 10240
DEG_STRIPE = N_PAD // NS
ROW_STRIPE = 640
LAST_STRIPE = N_NODES - (NS - 1) * ROW_STRIPE
ZROWS = 128

_mesh = plsc.VectorSubcoreMesh(
    core_axis_name="c", subcore_axis_name="s", num_cores=NC, num_subcores=NS
)


@functools.partial(
    pl.kernel,
    out_type=jax.ShapeDtypeStruct((NC, N_PAD), jnp.float32),
    mesh=_mesh,
    compiler_params=pltpu.CompilerParams(use_tc_tiling_on_sc=False),
    scratch_types=[
        pltpu.VMEM_SHARED((N_PAD,), jnp.float32),
        pltpu.VMEM((NB, EB), jnp.int32),
        pltpu.VMEM((128,), jnp.float32),
        pltpu.VMEM((DEG_STRIPE,), jnp.float32),
    ],
)
def _deg_kernel(dst_hbm, out_hbm, acc, dstbuf, ones, zbuf):
    cid = lax.axis_index("c")
    sid = lax.axis_index("s")
    wid = cid * NS + sid

    zero16 = jnp.zeros((16,), jnp.float32)
    one16 = jnp.ones((16,), jnp.float32)

    @pl.loop(0, DEG_STRIPE // 16)
    def _(i):
        zbuf[pl.ds(i * 16, 16)] = zero16

    @pl.loop(0, 8)
    def _(i):
        ones[pl.ds(i * 16, 16)] = one16

    pltpu.sync_copy(zbuf, acc.at[pl.ds(sid * DEG_STRIPE, DEG_STRIPE)])
    pltpu.sync_copy(dst_hbm.at[pl.ds(wid * NB, NB)], dstbuf)
    plsc.subcore_barrier()

    @pl.loop(0, OUTER)
    def _(i):
        for j in range(INNER):
            pltpu.sync_copy(ones.at[pl.ds(0, EB)],
                            acc.at[dstbuf.at[i * INNER + j]], add=True)

    plsc.subcore_barrier()

    pltpu.sync_copy(acc.at[pl.ds(sid * DEG_STRIPE, DEG_STRIPE)],
                    out_hbm.at[cid, pl.ds(sid * DEG_STRIPE, DEG_STRIPE)])


STRIPE = N_NODES // NS
GEB = 50
GNB = E_PER_TILE // GEB
NBUF = 4
GINNER = 4
GSTEADY = (GNB - NBUF) // GINNER


@functools.partial(
    pl.kernel,
    out_type=jax.ShapeDtypeStruct((NC, N_NODES, D), jnp.float32),
    mesh=_mesh,
    compiler_params=pltpu.CompilerParams(use_tc_tiling_on_sc=False),
    scratch_types=[
        pltpu.VMEM_SHARED((N_NODES, D), jnp.float32),
        pltpu.VMEM((GNB, GEB), jnp.int32),
        pltpu.VMEM((GNB, GEB), jnp.int32),
        pltpu.VMEM((NBUF, GEB, D), jnp.float32),
        [pltpu.SemaphoreType.DMA] * NBUF,
        pltpu.SemaphoreType.DMA,
        pltpu.SemaphoreType.DMA,
    ],
)
def _agg_kernel(h2_hbm, src_hbm, dst_hbm, out_hbm,
                acc, srcbuf, dstbuf, ringbuf, gsems, semi0, semi1):
    cid = lax.axis_index("c")
    sid = lax.axis_index("s")
    wid = cid * NS + sid

    zero16 = jnp.zeros((16,), jnp.float32)
    rows = [ringbuf.at[k] for k in range(NBUF)]

    ld_src = pltpu.async_copy(src_hbm.at[pl.ds(wid * GNB, GNB)], srcbuf, semi0)
    ld_dst = pltpu.async_copy(dst_hbm.at[pl.ds(wid * GNB, GNB)], dstbuf, semi1)

    @pl.loop(0, GEB)
    def _(i):
        for j in range(D // 16):
            ringbuf[0, i, pl.ds(j * 16, 16)] = zero16

    for k in range(STRIPE // GEB):
        pltpu.sync_copy(rows[0], acc.at[pl.ds(sid * STRIPE + k * GEB, GEB)])
    pltpu.sync_copy(rows[0].at[pl.ds(0, STRIPE % GEB)],
                    acc.at[pl.ds(sid * STRIPE + STRIPE - STRIPE % GEB,
                                 STRIPE % GEB)])

    ld_src.wait()
    for k in range(NBUF - 1):
        pltpu.async_copy(h2_hbm.at[srcbuf.at[k]], rows[k], gsems[k])
    ld_dst.wait()
    plsc.subcore_barrier()

    def block(b, p, prefetch):
        pltpu.make_async_copy(h2_hbm.at[srcbuf.at[b]], rows[p], gsems[p]).wait()
        if prefetch:
            q = (p + NBUF - 1) % NBUF
            pltpu.async_copy(h2_hbm.at[srcbuf.at[b + NBUF - 1]],
                             rows[q], gsems[q])
        pltpu.sync_copy(rows[p], acc.at[dstbuf.at[b]], add=True)

    @pl.loop(0, GSTEADY)
    def _(i):
        for j in range(GINNER):
            block(i * GINNER + j, j, True)

    for j in range(NBUF):
        b = GSTEADY * GINNER + j
        block(b, b % NBUF, j == 0)

    plsc.subcore_barrier()

    for k in range(STRIPE // GEB):
        r0 = sid * STRIPE + k * GEB
        pltpu.sync_copy(acc.at[pl.ds(r0, GEB)],
                        out_hbm.at[cid, pl.ds(r0, GEB)])
    r0 = sid * STRIPE + STRIPE - STRIPE % GEB
    pltpu.sync_copy(acc.at[pl.ds(r0, STRIPE % GEB)],
                    out_hbm.at[cid, pl.ds(r0, STRIPE % GEB)])


ROWS_BLK = 1024
GRID = (N_NODES + ROWS_BLK - 1) // ROWS_BLK


def _mm_body(x_ref, w_ref, h_ref):
    h_ref[...] = jnp.dot(x_ref[...], w_ref[...],
                         preferred_element_type=jnp.float32)


def _matmul(x, W):
    return pl.pallas_call(
        _mm_body,
        grid=(GRID,),
        in_specs=[
            pl.BlockSpec((ROWS_BLK, D), lambda i: (i, 0)),
            pl.BlockSpec((D, D), lambda i: (0, 0)),
        ],
        out_specs=pl.BlockSpec((ROWS_BLK, D), lambda i: (i, 0)),
        out_shape=jax.ShapeDtypeStruct((N_NODES, D), jnp.float32),
    )(x, W)


def _scale_body(h_ref, degp_ref, h2_ref):
    deg = degp_ref[0, :] + degp_ref[1, :] + 1.0
    h2_ref[...] = h_ref[...] * lax.rsqrt(deg)[:, None]


def _scale(h, degp):
    return pl.pallas_call(
        _scale_body,
        grid=(GRID,),
        in_specs=[
            pl.BlockSpec((ROWS_BLK, D), lambda i: (i, 0)),
            pl.BlockSpec((NC, ROWS_BLK), lambda i: (0, i)),
        ],
        out_specs=pl.BlockSpec((ROWS_BLK, D), lambda i: (i, 0)),
        out_shape=jax.ShapeDtypeStruct((N_NODES, D), jnp.float32),
    )(h, degp)


def _fin_body(s_ref, h2_ref, degp_ref, b_ref, out_ref):
    deg = degp_ref[0, :] + degp_ref[1, :] + 1.0
    dis = lax.rsqrt(deg)
    tot = s_ref[0] + s_ref[1] + h2_ref[...]
    out_ref[...] = jnp.maximum(tot * dis[:, None] + b_ref[...], 0.0)


def _finalize(S, h2, degp, b2d):
    return pl.pallas_call(
        _fin_body,
        grid=(GRID,),
        in_specs=[
            pl.BlockSpec((NC, ROWS_BLK, D), lambda i: (0, i, 0)),
            pl.BlockSpec((ROWS_BLK, D), lambda i: (i, 0)),
            pl.BlockSpec((NC, ROWS_BLK), lambda i: (0, i)),
            pl.BlockSpec((1, D), lambda i: (0, 0)),
        ],
        out_specs=pl.BlockSpec((ROWS_BLK, D), lambda i: (i, 0)),
        out_shape=jax.ShapeDtypeStruct((N_NODES, D), jnp.float32),
    )(S, h2, degp, b2d)


def kernel(x, edge_index, W, b):
    src = edge_index[0].astype(jnp.int32)
    dst = edge_index[1].astype(jnp.int32)
    degp = _deg_kernel(dst.reshape(NW * NB, EB))
    h = _matmul(x, W)
    h2 = _scale(h, degp)
    S = _agg_kernel(h2, src.reshape(NW * GNB, GEB), dst.reshape(NW * GNB, GEB))
    return _finalize(S, h2, degp, b.reshape(1, D))


# speedup vs baseline: 1.1393x; 1.1393x over previous
import functools

import jax
import jax.numpy as jnp
from jax import lax
from jax.experimental import pallas as pl
from jax.experimental.pallas import tpu as pltpu
from jax.experimental.pallas import tpu_sc as plsc

N_NODES = 10000
D = 128
N_EDGES = 320000

NC = 2
NS = 16
NW = NC * NS
E_PER_TILE = N_EDGES // NW
EB = 125
NB = E_PER_TILE // EB
INNER = 5
OUTER = NB // INNER

N_PAD = 10240
DEG_STRIPE = N_PAD // NS
ROW_STRIPE = 640
LAST_STRIPE = N_NODES - (NS - 1) * ROW_STRIPE
ZROWS = 128

_mesh = plsc.VectorSubcoreMesh(
    core_axis_name="c", subcore_axis_name="s", num_cores=NC, num_subcores=NS
)


@functools.partial(
    pl.kernel,
    out_type=jax.ShapeDtypeStruct((NC, N_PAD), jnp.float32),
    mesh=_mesh,
    compiler_params=pltpu.CompilerParams(use_tc_tiling_on_sc=False),
    scratch_types=[
        pltpu.VMEM_SHARED((N_PAD,), jnp.float32),
        pltpu.VMEM((NB, EB), jnp.int32),
        pltpu.VMEM((128,), jnp.float32),
        pltpu.VMEM((DEG_STRIPE,), jnp.float32),
    ],
)
def _deg_kernel(dst_hbm, out_hbm, acc, dstbuf, ones, zbuf):
    cid = lax.axis_index("c")
    sid = lax.axis_index("s")
    wid = cid * NS + sid

    zero16 = jnp.zeros((16,), jnp.float32)
    one16 = jnp.ones((16,), jnp.float32)

    @pl.loop(0, DEG_STRIPE // 16)
    def _(i):
        zbuf[pl.ds(i * 16, 16)] = zero16

    @pl.loop(0, 8)
    def _(i):
        ones[pl.ds(i * 16, 16)] = one16

    pltpu.sync_copy(zbuf, acc.at[pl.ds(sid * DEG_STRIPE, DEG_STRIPE)])
    pltpu.sync_copy(dst_hbm.at[pl.ds(wid * NB, NB)], dstbuf)
    plsc.subcore_barrier()

    @pl.loop(0, OUTER)
    def _(i):
        for j in range(INNER):
            pltpu.sync_copy(ones.at[pl.ds(0, EB)],
                            acc.at[dstbuf.at[i * INNER + j]], add=True)

    plsc.subcore_barrier()

    pltpu.sync_copy(acc.at[pl.ds(sid * DEG_STRIPE, DEG_STRIPE)],
                    out_hbm.at[cid, pl.ds(sid * DEG_STRIPE, DEG_STRIPE)])


STRIPE = N_NODES // NS
GEB = 40
GNB = E_PER_TILE // GEB
NBUF = 5
GINNER = 5
GSTEADY = (GNB - NBUF) // GINNER


@functools.partial(
    pl.kernel,
    out_type=jax.ShapeDtypeStruct((NC, N_NODES, D), jnp.float32),
    mesh=_mesh,
    compiler_params=pltpu.CompilerParams(use_tc_tiling_on_sc=False),
    scratch_types=[
        pltpu.VMEM_SHARED((N_NODES, D), jnp.float32),
        pltpu.VMEM((GNB, GEB), jnp.int32),
        pltpu.VMEM((GNB, GEB), jnp.int32),
        pltpu.VMEM((NBUF, GEB, D), jnp.float32),
        [pltpu.SemaphoreType.DMA] * NBUF,
        pltpu.SemaphoreType.DMA,
        pltpu.SemaphoreType.DMA,
    ],
)
def _agg_kernel(h2_hbm, src_hbm, dst_hbm, out_hbm,
                acc, srcbuf, dstbuf, ringbuf, gsems, semi0, semi1):
    cid = lax.axis_index("c")
    sid = lax.axis_index("s")
    wid = cid * NS + sid

    zero16 = jnp.zeros((16,), jnp.float32)
    rows = [ringbuf.at[k] for k in range(NBUF)]

    ld_src = pltpu.async_copy(src_hbm.at[pl.ds(wid * GNB, GNB)], srcbuf, semi0)
    ld_dst = pltpu.async_copy(dst_hbm.at[pl.ds(wid * GNB, GNB)], dstbuf, semi1)

    @pl.loop(0, GEB)
    def _(i):
        for j in range(D // 16):
            ringbuf[0, i, pl.ds(j * 16, 16)] = zero16

    for k in range(STRIPE // GEB):
        pltpu.sync_copy(rows[0], acc.at[pl.ds(sid * STRIPE + k * GEB, GEB)])
    pltpu.sync_copy(rows[0].at[pl.ds(0, STRIPE % GEB)],
                    acc.at[pl.ds(sid * STRIPE + STRIPE - STRIPE % GEB,
                                 STRIPE % GEB)])

    ld_src.wait()
    for k in range(NBUF - 1):
        pltpu.async_copy(h2_hbm.at[srcbuf.at[k]], rows[k], gsems[k])
    ld_dst.wait()
    plsc.subcore_barrier()

    def block(b, p, prefetch):
        pltpu.make_async_copy(h2_hbm.at[srcbuf.at[b]], rows[p], gsems[p]).wait()
        if prefetch:
            q = (p + NBUF - 1) % NBUF
            pltpu.async_copy(h2_hbm.at[srcbuf.at[b + NBUF - 1]],
                             rows[q], gsems[q])
        pltpu.sync_copy(rows[p], acc.at[dstbuf.at[b]], add=True)

    @pl.loop(0, GSTEADY)
    def _(i):
        for j in range(GINNER):
            block(i * GINNER + j, j, True)

    for j in range(NBUF):
        b = GSTEADY * GINNER + j
        block(b, b % NBUF, j == 0)

    plsc.subcore_barrier()

    for k in range(STRIPE // GEB):
        r0 = sid * STRIPE + k * GEB
        pltpu.sync_copy(acc.at[pl.ds(r0, GEB)],
                        out_hbm.at[cid, pl.ds(r0, GEB)])
    r0 = sid * STRIPE + STRIPE - STRIPE % GEB
    pltpu.sync_copy(acc.at[pl.ds(r0, STRIPE % GEB)],
                    out_hbm.at[cid, pl.ds(r0, STRIPE % GEB)])


ROWS_BLK = 1024
GRID = (N_NODES + ROWS_BLK - 1) // ROWS_BLK


def _mm_body(x_ref, w_ref, degp_ref, h2_ref):
    deg = degp_ref[0, :] + degp_ref[1, :] + 1.0
    dis = lax.rsqrt(deg)
    h = jnp.dot(x_ref[...], w_ref[...], preferred_element_type=jnp.float32)
    h2_ref[...] = h * dis[:, None]


def _matmul_scale(x, W, degp):
    return pl.pallas_call(
        _mm_body,
        grid=(GRID,),
        in_specs=[
            pl.BlockSpec((ROWS_BLK, D), lambda i: (i, 0)),
            pl.BlockSpec((D, D), lambda i: (0, 0)),
            pl.BlockSpec((NC, ROWS_BLK), lambda i: (0, i)),
        ],
        out_specs=pl.BlockSpec((ROWS_BLK, D), lambda i: (i, 0)),
        out_shape=jax.ShapeDtypeStruct((N_NODES, D), jnp.float32),
    )(x, W, degp)


def _fin_body(s_ref, h2_ref, degp_ref, b_ref, out_ref):
    deg = degp_ref[0, :] + degp_ref[1, :] + 1.0
    dis = lax.rsqrt(deg)
    tot = s_ref[0] + s_ref[1] + h2_ref[...]
    out_ref[...] = jnp.maximum(tot * dis[:, None] + b_ref[...], 0.0)


def _finalize(S, h2, degp, b2d):
    return pl.pallas_call(
        _fin_body,
        grid=(GRID,),
        in_specs=[
            pl.BlockSpec((NC, ROWS_BLK, D), lambda i: (0, i, 0)),
            pl.BlockSpec((ROWS_BLK, D), lambda i: (i, 0)),
            pl.BlockSpec((NC, ROWS_BLK), lambda i: (0, i)),
            pl.BlockSpec((1, D), lambda i: (0, 0)),
        ],
        out_specs=pl.BlockSpec((ROWS_BLK, D), lambda i: (i, 0)),
        out_shape=jax.ShapeDtypeStruct((N_NODES, D), jnp.float32),
    )(S, h2, degp, b2d)


def kernel(x, edge_index, W, b):
    src = edge_index[0].astype(jnp.int32)
    dst = edge_index[1].astype(jnp.int32)
    degp = _deg_kernel(dst.reshape(NW * NB, EB))
    h2 = _matmul_scale(x, W, degp)
    S = _agg_kernel(h2, src.reshape(NW * GNB, GEB), dst.reshape(NW * GNB, GEB))
    return _finalize(S, h2, degp, b.reshape(1, D))
